# baseline (speedup 1.0000x reference)
import functools

import jax
import jax.numpy as jnp
from jax import lax
from jax.experimental import pallas as pl
from jax.experimental.pallas import tpu as pltpu
from jax.experimental.pallas import tpu_sc as plsc

_BATCH = 16384
_FEAT = 128
_NC = 2
_NS = 16
_NW = _NC * _NS
_ROWS_PER_W = _BATCH // _NW
_CHUNK = 64
_NCHUNK = _ROWS_PER_W // _CHUNK
_LANES = 16
_VPF = _FEAT // _LANES


def _make_sc_loss():
    mesh = plsc.VectorSubcoreMesh(core_axis_name="c", subcore_axis_name="s")

    @functools.partial(
        pl.kernel,
        mesh=mesh,
        compiler_params=pltpu.CompilerParams(needs_layout_passes=False),
        out_type=jax.ShapeDtypeStruct((_NW, _LANES), jnp.float32),
        scratch_types=[
            pltpu.VMEM((_NCHUNK, _CHUNK), jnp.int32),
            pltpu.VMEM((2, _CHUNK, _FEAT), jnp.float32),
            pltpu.VMEM((2, _CHUNK, _FEAT), jnp.float32),
            pltpu.VMEM((_LANES,), jnp.float32),
            pltpu.SemaphoreType.DMA,
            pltpu.SemaphoreType.DMA,
            pltpu.SemaphoreType.DMA,
            pltpu.SemaphoreType.DMA,
        ],
    )
    def sc_loss(x_hbm, lab_hbm, cen_hbm, out_hbm,
                idx_v, xb, cb, res_v, sx0, sx1, sc0, sc1):
        wid = lax.axis_index("s") * _NC + lax.axis_index("c")
        sems_x = (sx0, sx1)
        sems_c = (sc0, sc1)
        hx = {}
        hc = {}

        def start_x(i):
            b = i % 2
            hx[i] = pltpu.async_copy(x_hbm.at[wid, i], xb.at[b], sems_x[b])

        def start_c(i):
            b = i % 2
            hc[i] = pltpu.async_copy(cen_hbm.at[idx_v.at[i]], cb.at[b], sems_c[b])

        start_x(0)
        pltpu.sync_copy(lab_hbm.at[wid], idx_v)
        start_c(0)
        total_vec = jnp.zeros((_LANES,), jnp.float32)
        for i in range(_NCHUNK):
            if i + 1 < _NCHUNK:
                start_x(i + 1)
                start_c(i + 1)
            hx[i].wait()
            hc[i].wait()
            b = i % 2

            def row_body(r, tv, b=b):
                acc = None
                for f in range(_VPF):
                    xv = xb[b, r, pl.ds(_LANES * f, _LANES)]
                    cv = cb[b, r, pl.ds(_LANES * f, _LANES)]
                    d = xv - cv
                    acc = d * d if acc is None else acc + d * d
                cs = plsc.cumsum(acc)
                cs = jnp.minimum(jnp.maximum(cs, jnp.float32(1e-12)),
                                 jnp.float32(1e12))
                return tv + cs

            total_vec = plsc.parallel_loop(
                0, _CHUNK, step=1, unroll=4, carry=total_vec)(row_body)
        res_v[...] = total_vec
        pltpu.sync_copy(res_v, out_hbm.at[wid])

    return sc_loss


_sc_loss = _make_sc_loss()


def kernel(x, labels, centers):
    x4 = x.reshape(_NW, _NCHUNK, _CHUNK, _FEAT)
    lab3 = labels.astype(jnp.int32).reshape(_NW, _NCHUNK, _CHUNK)
    partials = _sc_loss(x4, lab3, centers)
    return jnp.sum(partials[:, _LANES - 1]) / jnp.float32(_BATCH)


# speedup vs baseline: 1.0514x; 1.0514x over previous
import functools

import jax
import jax.numpy as jnp
from jax import lax
from jax.experimental import pallas as pl
from jax.experimental.pallas import tpu as pltpu
from jax.experimental.pallas import tpu_sc as plsc

_BATCH = 16384
_FEAT = 128
_NC = 2
_NS = 16
_NW = _NC * _NS
_ROWS_PER_W = _BATCH // _NW
_CHUNK = 128
_NCHUNK = _ROWS_PER_W // _CHUNK
_LANES = 16
_VPF = _FEAT // _LANES


def _make_sc_loss():
    mesh = plsc.VectorSubcoreMesh(core_axis_name="c", subcore_axis_name="s")

    @functools.partial(
        pl.kernel,
        mesh=mesh,
        compiler_params=pltpu.CompilerParams(needs_layout_passes=False),
        out_type=jax.ShapeDtypeStruct((_NW, _LANES), jnp.float32),
        scratch_types=[
            pltpu.VMEM((_NCHUNK, _CHUNK), jnp.int32),
            pltpu.VMEM((2, _CHUNK, _FEAT), jnp.float32),
            pltpu.VMEM((2, _CHUNK, _FEAT), jnp.float32),
            pltpu.VMEM((_LANES,), jnp.float32),
            pltpu.SemaphoreType.DMA,
            pltpu.SemaphoreType.DMA,
            pltpu.SemaphoreType.DMA,
            pltpu.SemaphoreType.DMA,
        ],
    )
    def sc_loss(x_hbm, lab_hbm, cen_hbm, out_hbm,
                idx_v, xb, cb, res_v, sx0, sx1, sc0, sc1):
        wid = lax.axis_index("s") * _NC + lax.axis_index("c")
        sems_x = (sx0, sx1)
        sems_c = (sc0, sc1)
        hx = {}
        hc = {}

        def start_x(i):
            b = i % 2
            hx[i] = pltpu.async_copy(x_hbm.at[wid, i], xb.at[b], sems_x[b])

        def start_c(i):
            b = i % 2
            hc[i] = pltpu.async_copy(cen_hbm.at[idx_v.at[i]], cb.at[b], sems_c[b])

        start_x(0)
        pltpu.sync_copy(lab_hbm.at[wid], idx_v)
        start_c(0)
        total_vec = jnp.zeros((_LANES,), jnp.float32)
        for i in range(_NCHUNK):
            if i + 1 < _NCHUNK:
                start_x(i + 1)
                start_c(i + 1)
            hx[i].wait()
            hc[i].wait()
            b = i % 2

            def row_body(r, tv, b=b):
                acc = None
                for f in range(_VPF):
                    xv = xb[b, r, pl.ds(_LANES * f, _LANES)]
                    cv = cb[b, r, pl.ds(_LANES * f, _LANES)]
                    d = xv - cv
                    acc = d * d if acc is None else acc + d * d
                cs = plsc.cumsum(acc)
                cs = jnp.minimum(jnp.maximum(cs, jnp.float32(1e-12)),
                                 jnp.float32(1e12))
                return tv + cs

            total_vec = plsc.parallel_loop(
                0, _CHUNK, step=1, unroll=4, carry=total_vec)(row_body)
        res_v[...] = total_vec
        pltpu.sync_copy(res_v, out_hbm.at[wid])

    return sc_loss


_sc_loss = _make_sc_loss()


def kernel(x, labels, centers):
    x4 = x.reshape(_NW, _NCHUNK, _CHUNK, _FEAT)
    lab3 = labels.astype(jnp.int32).reshape(_NW, _NCHUNK, _CHUNK)
    partials = _sc_loss(x4, lab3, centers)
    return jnp.sum(partials[:, _LANES - 1]) / jnp.float32(_BATCH)


# speedup vs baseline: 1.0561x; 1.0045x over previous
import functools

import jax
import jax.numpy as jnp
from jax import lax
from jax.experimental import pallas as pl
from jax.experimental.pallas import tpu as pltpu
from jax.experimental.pallas import tpu_sc as plsc

_BATCH = 16384
_FEAT = 128
_NC = 2
_NS = 16
_NW = _NC * _NS
_ROWS_PER_W = _BATCH // _NW
_CHUNKS = (64, 128, 128, 128, 64)
_LANES = 16
_VPF = _FEAT // _LANES
_BUF = 128


def _make_sc_loss():
    mesh = plsc.VectorSubcoreMesh(core_axis_name="c", subcore_axis_name="s")

    @functools.partial(
        pl.kernel,
        mesh=mesh,
        compiler_params=pltpu.CompilerParams(needs_layout_passes=False),
        out_type=jax.ShapeDtypeStruct((_NW, _LANES), jnp.float32),
        scratch_types=[
            pltpu.VMEM((_ROWS_PER_W,), jnp.int32),
            pltpu.VMEM((2, _BUF, _FEAT), jnp.float32),
            pltpu.VMEM((2, _BUF, _FEAT), jnp.float32),
            pltpu.VMEM((_LANES,), jnp.float32),
            pltpu.SemaphoreType.DMA,
            pltpu.SemaphoreType.DMA,
            pltpu.SemaphoreType.DMA,
            pltpu.SemaphoreType.DMA,
        ],
    )
    def sc_loss(x_hbm, lab_hbm, cen_hbm, out_hbm,
                idx_v, xb, cb, res_v, sx0, sx1, sc0, sc1):
        wid = lax.axis_index("s") * _NC + lax.axis_index("c")
        sems_x = (sx0, sx1)
        sems_c = (sc0, sc1)
        starts = [sum(_CHUNKS[:i]) for i in range(len(_CHUNKS))]
        hx = {}
        hc = {}

        def start_x(i):
            b = i % 2
            hx[i] = pltpu.async_copy(
                x_hbm.at[wid, pl.ds(starts[i], _CHUNKS[i])],
                xb.at[b, pl.ds(0, _CHUNKS[i])], sems_x[b])

        def start_c(i):
            b = i % 2
            hc[i] = pltpu.async_copy(
                cen_hbm.at[idx_v.at[pl.ds(starts[i], _CHUNKS[i])]],
                cb.at[b, pl.ds(0, _CHUNKS[i])], sems_c[b])

        start_x(0)
        pltpu.sync_copy(lab_hbm.at[wid], idx_v)
        start_c(0)
        total_vec = jnp.zeros((_LANES,), jnp.float32)
        for i in range(len(_CHUNKS)):
            if i + 1 < len(_CHUNKS):
                start_x(i + 1)
                start_c(i + 1)
            hx[i].wait()
            hc[i].wait()
            b = i % 2

            def row_body(r, tv, b=b):
                acc = None
                for f in range(_VPF):
                    xv = xb[b, r, pl.ds(_LANES * f, _LANES)]
                    cv = cb[b, r, pl.ds(_LANES * f, _LANES)]
                    d = xv - cv
                    acc = d * d if acc is None else acc + d * d
                cs = plsc.cumsum(acc)
                cs = jnp.minimum(jnp.maximum(cs, jnp.float32(1e-12)),
                                 jnp.float32(1e12))
                return tv + cs

            total_vec = plsc.parallel_loop(
                0, _CHUNKS[i], step=1, unroll=4, carry=total_vec)(row_body)
        res_v[...] = total_vec
        pltpu.sync_copy(res_v, out_hbm.at[wid])

    return sc_loss


_sc_loss = _make_sc_loss()


def kernel(x, labels, centers):
    x3 = x.reshape(_NW, _ROWS_PER_W, _FEAT)
    lab2 = labels.astype(jnp.int32).reshape(_NW, _ROWS_PER_W)
    partials = _sc_loss(x3, lab2, centers)
    return jnp.sum(partials[:, _LANES - 1]) / jnp.float32(_BATCH)
